# baseline (speedup 1.0000x reference)
import functools

import jax
import jax.numpy as jnp
from jax import lax
from jax.experimental import pallas as pl
from jax.experimental.pallas import tpu as pltpu
from jax.experimental.pallas import tpu_sc as plsc

_CLASS_NUM = 1000000
_FEATURE_DIM = 64
_BATCH = 16384
_LAMADA = 0.01

_NC = 2
_NS = 16
_NW = _NC * _NS

_SB = _BATCH // _NW
_HB = _BATCH // _NS
_CH = 128
_G_CHUNKS = _SB // _CH
_H_CHUNKS = _HB // _CH


def _sc_body(tag_hbm, center_hbm, crows_hbm, counts_hbm, *scr):
    gidx = scr[0:_G_CHUNKS]
    hidx = scr[_G_CHUNKS:_G_CHUNKS + _H_CHUNKS]
    k = _G_CHUNKS + _H_CHUNKS
    cnt = scr[k:k + _G_CHUNKS]
    rows = scr[k + _G_CHUNKS]
    ones = scr[k + _G_CHUNKS + 1]
    zeros = scr[k + _G_CHUNKS + 2]
    hist = scr[k + _G_CHUNKS + 3]
    sem = scr[k + _G_CHUNKS + 4]

    c = lax.axis_index("c")
    s = lax.axis_index("s")
    wid = c * _NS + s
    base = wid * _SB
    hbase = s * _HB

    copies = []
    for j in range(_G_CHUNKS):
        pltpu.sync_copy(tag_hbm.at[pl.ds(base + j * _CH, _CH)], gidx[j])
    for j in range(_G_CHUNKS):
        copies.append(
            pltpu.async_copy(
                center_hbm.at[gidx[j]], rows.at[pl.ds(j * _CH, _CH)], sem
            )
        )

    for j in range(_H_CHUNKS):
        pltpu.sync_copy(tag_hbm.at[pl.ds(hbase + j * _CH, _CH)], hidx[j])
    for i in range(_CH // 16):
        ones[pl.ds(i * 16, 16)] = jnp.full((16,), 1, jnp.int32)
        zeros[pl.ds(i * 16, 16)] = jnp.full((16,), 0, jnp.int32)

    for j in range(_H_CHUNKS):
        pltpu.sync_copy(zeros, hist.at[hidx[j]])
    plsc.subcore_barrier()

    for j in range(_H_CHUNKS):
        pltpu.sync_copy(ones, hist.at[hidx[j]], add=True)
    plsc.subcore_barrier()

    for j in range(_G_CHUNKS):
        pltpu.sync_copy(hist.at[gidx[j]], cnt[j])
    for j in range(_G_CHUNKS):
        pltpu.sync_copy(cnt[j], counts_hbm.at[pl.ds(base + j * _CH, _CH)])

    for cp in copies:
        cp.wait()
    pltpu.sync_copy(rows, crows_hbm.at[pl.ds(base, _SB)])


@jax.jit
def _sc_gather_counts(tag, center):
    mesh = plsc.VectorSubcoreMesh(core_axis_name="c", subcore_axis_name="s")
    scratch = (
        [pltpu.VMEM((_CH,), jnp.int32) for _ in range(_G_CHUNKS)]
        + [pltpu.VMEM((_CH,), jnp.int32) for _ in range(_H_CHUNKS)]
        + [pltpu.VMEM((_CH,), jnp.int32) for _ in range(_G_CHUNKS)]
        + [
            pltpu.VMEM((_SB, _FEATURE_DIM), jnp.float32),
            pltpu.VMEM((_CH,), jnp.int32),
            pltpu.VMEM((_CH,), jnp.int32),
            pltpu.VMEM_SHARED((_CLASS_NUM,), jnp.int32),
            pltpu.SemaphoreType.DMA,
        ]
    )
    fn = pl.kernel(
        _sc_body,
        out_type=(
            jax.ShapeDtypeStruct((_BATCH, _FEATURE_DIM), jnp.float32),
            jax.ShapeDtypeStruct((_BATCH,), jnp.int32),
        ),
        mesh=mesh,
        scratch_types=scratch,
        compiler_params=pltpu.CompilerParams(use_tc_tiling_on_sc=False),
    )
    return fn(tag, center)


def _tc_body(f_ref, c_ref, n_ref, o_ref):
    diff = f_ref[:] - c_ref[:]
    sq = jnp.sum(diff * diff, axis=1, keepdims=True)
    d = jnp.sqrt(sq)
    n = n_ref[:].astype(jnp.float32)
    o_ref[0, 0] = jnp.sum(d / n) * (0.5 * _LAMADA)


@jax.jit
def _tc_combine(feature, crows, counts):
    out = pl.pallas_call(
        _tc_body,
        out_shape=jax.ShapeDtypeStruct((1, 1), jnp.float32),
        out_specs=pl.BlockSpec(memory_space=pltpu.SMEM),
    )(feature, crows, counts.reshape(_BATCH, 1))
    return out[0, 0]


def kernel(tag, feature, center):
    tag = tag.astype(jnp.int32)
    crows, counts = _sc_gather_counts(tag, center)
    return _tc_combine(feature, crows, counts)
